# Initial kernel scaffold; baseline (speedup 1.0000x reference)
import jax
import jax.numpy as jnp
from jax.experimental import pallas as pl


def kernel(boxes3D, classification, poses, confidence):
    raise NotImplementedError("write your pallas kernel here")


# speedup vs baseline: 1.7343x; 1.7343x over previous
import jax
import jax.numpy as jnp
from jax import lax
from jax.experimental import pallas as pl
from jax.experimental.pallas import tpu as pltpu
from jax.experimental.pallas import tpu_sc as plsc

NUM_CLASSES = 15
K = 300
NFLAT = 5000 * NUM_CLASSES
NW = 16
CHUNK = 4704
NPAD = NW * CHUNK
NV = CHUNK // 16
CCAP = 128
CUSE = 64
DENSE = NW * CUSE
SELCAP = 128
MAXOUT = 512
OSL = MAXOUT // NW
PW = OSL * 12
BIAS = 0x3F000000
KEYMAX = 0x7FFFFF


def _popcnt(mask):
    v = plsc.all_reduce_population_count(mask)
    return jnp.max(v) if v.ndim else v


def _suffix_search(gath, kthr):
    z = jnp.zeros((16,), jnp.int32)
    carryv, bv, mv = z, z, z
    for j in reversed(range(16)):
        tot = z
        for r in range(NW):
            tot = tot + gath[r, pl.ds(j * 16, 16)]
        cs = plsc.cumsum(lax.rev(tot, (0,)))
        s_incl = lax.rev(cs, (0,)) + carryv
        g = s_incl - tot
        ge = g >= kthr
        bv = bv + plsc.all_reduce_population_count(ge)
        mv = jnp.maximum(mv, jnp.where(ge, 0, g))
        carryv = carryv + jnp.max(cs)
    return bv, mv


def _histogram(keys_v, hist_f, lane, bucket_fn, mask_fn):
    def zbody(i, _):
        hist_f[pl.ds(i * 16, 16)] = jnp.zeros((16,), jnp.int32)
        return 0
    lax.fori_loop(0, NW * 256 // 16, zbody, 0)
    ones = jnp.full((16,), 1, jnp.int32)
    row = lane * 256

    def hbody(i, _):
        keyv = keys_v[pl.ds(i * 16, 16)]
        plsc.addupdate_scatter(hist_f, [row + bucket_fn(keyv)], ones,
                               mask=mask_fn(keyv))
        return 0
    lax.fori_loop(0, NV, hbody, 0)


def _reduce_hist(hist_f, red_v):
    def rbody(c, _):
        acc = jnp.zeros((16,), jnp.int32)
        for r in range(NW):
            acc = acc + hist_f[pl.ds(r * 256 + c * 16, 16)]
        red_v[pl.ds(c * 16, 16)] = acc
        return 0
    lax.fori_loop(0, 16, rbody, 0)


def _body(scores_hbm, poses_hbm,
          oscore_hbm, olabel_hbm, oposes_hbm, obox_hbm,
          chunk_v, keys_v, hist_f, red_v, gath_v,
          cand_k, cand_x, allk_v, allx_v, cnt2_v, tmp_v,
          dense_k, dense_x,
          sel_r, sel_s, sel_l, sel_b, sel_w,
          asel_r, asel_s, asel_l, asel_b, asel_w,
          o_score, o_label, o_box, o_row, pidx_v, prow_v,
          sh_hist, sh_cnt, sh_ck, sh_cx,
          sh_selr, sh_sels, sh_sell, sh_selb, sh_selw, sem):
    w = lax.axis_index("s")
    lane = lax.iota(jnp.int32, 16)
    base = w * CHUNK
    kvec = jnp.full((16,), K, jnp.int32)
    zero16 = jnp.zeros((16,), jnp.int32)

    pltpu.sync_copy(scores_hbm.at[pl.ds(base, CHUNK)], chunk_v)

    def keybody(i, _):
        sv = chunk_v[pl.ds(i * 16, 16)]
        bits = lax.bitcast_convert_type(sv, jnp.int32)
        validm = sv > 0.5
        keyv = jnp.where(
            validm, jnp.clip(bits - BIAS, 1, KEYMAX), 0)
        keys_v[pl.ds(i * 16, 16)] = keyv
        return 0
    lax.fori_loop(0, NV, keybody, 0)

    _histogram(keys_v, hist_f, lane,
               lambda kv: kv >> 15, lambda kv: kv > 0)
    _reduce_hist(hist_f, red_v)
    pltpu.sync_copy(red_v, sh_hist.at[w])
    plsc.subcore_barrier()

    pltpu.sync_copy(sh_hist, gath_v)
    b1v, m1v = _suffix_search(gath_v, kvec)

    _histogram(keys_v, hist_f, lane,
               lambda kv: (kv >> 7) & 255,
               lambda kv: (kv > 0) & ((kv >> 15) == b1v))
    _reduce_hist(hist_f, red_v)
    pltpu.sync_copy(red_v, sh_hist.at[w])
    plsc.subcore_barrier()
    pltpu.sync_copy(sh_hist, gath_v)
    b2v, _ = _suffix_search(gath_v, kvec - m1v)

    ktv = jnp.maximum((b1v << 15) | (b2v << 7), 1)

    for v in range(CCAP // 16):
        cand_k[pl.ds(v * 16, 16)] = zero16
        cand_x[pl.ds(v * 16, 16)] = zero16

    def cbody(i, off):
        keyv = keys_v[pl.ds(i * 16, 16)]
        m = keyv >= ktv
        idxv = base + i * 16 + lane
        offc = jnp.minimum(off, CUSE - 16)
        plsc.store_compressed(cand_k.at[pl.ds(offc, 16)], keyv, mask=m)
        plsc.store_compressed(cand_x.at[pl.ds(offc, 16)], idxv, mask=m)
        return off + _popcnt(m)
    myc = lax.fori_loop(0, NV, cbody, jnp.int32(0))
    myc = jnp.minimum(myc, CUSE)

    mycv = jnp.full((16,), myc, jnp.int32)
    for v in range(CCAP // 16):
        tmp_v[pl.ds(v * 16, 16)] = mycv
    pltpu.sync_copy(tmp_v, sh_cnt.at[w])
    pltpu.sync_copy(cand_k, sh_ck.at[w])
    pltpu.sync_copy(cand_x, sh_cx.at[w])
    plsc.subcore_barrier()

    pltpu.sync_copy(sh_ck, allk_v)
    pltpu.sync_copy(sh_cx, allx_v)
    pltpu.sync_copy(sh_cnt, cnt2_v)

    ctot = jnp.int32(0)
    for r in range(NW):
        cntr = jnp.max(cnt2_v[r, pl.ds(0, 16)])
        for v in range(CUSE // 16):
            m = (v * 16 + lane) < cntr
            kk = allk_v[r, pl.ds(v * 16, 16)]
            xx = allx_v[r, pl.ds(v * 16, 16)]
            offc = jnp.minimum(ctot, DENSE - 16)
            plsc.store_compressed(dense_k.at[pl.ds(offc, 16)], kk, mask=m)
            plsc.store_compressed(dense_x.at[pl.ds(offc, 16)], xx, mask=m)
            ctot = ctot + _popcnt(m)

    own_k = [cand_k[pl.ds(v * 16, 16)] for v in range(CUSE // 16)]
    own_x = [cand_x[pl.ds(v * 16, 16)] for v in range(CUSE // 16)]

    def rjbody(j, ranks):
        jj = jnp.full((16,), j, jnp.int32)
        kj = plsc.load_gather(dense_k, [jj])
        xj = plsc.load_gather(dense_x, [jj])
        out = []
        for v in range(CUSE // 16):
            beat = (kj > own_k[v]) | ((kj == own_k[v]) & (xj < own_x[v]))
            out.append(ranks[v] + jnp.where(beat, 1, 0))
        return tuple(out)
    ranks = lax.fori_loop(0, ctot, rjbody,
                          tuple(zero16 for _ in range(CUSE // 16)))

    for v in range(SELCAP // 16):
        sel_r[pl.ds(v * 16, 16)] = K + v * 16 + lane

    selcnt = jnp.int32(0)
    for v in range(CUSE // 16):
        selm = (ranks[v] < kvec) & (own_k[v] > 0)
        sc = jnp.minimum(selcnt, SELCAP - 16)
        scorev = lax.bitcast_convert_type(own_k[v] + BIAS, jnp.float32)
        plsc.store_compressed(sel_r.at[pl.ds(sc, 16)], ranks[v], mask=selm)
        plsc.store_compressed(sel_s.at[pl.ds(sc, 16)], scorev, mask=selm)
        plsc.store_compressed(sel_l.at[pl.ds(sc, 16)],
                              own_x[v] % NUM_CLASSES, mask=selm)
        plsc.store_compressed(sel_b.at[pl.ds(sc, 16)],
                              own_x[v] // NUM_CLASSES, mask=selm)
        plsc.store_compressed(sel_w.at[pl.ds(sc, 16)], own_x[v], mask=selm)
        selcnt = selcnt + _popcnt(selm)

    pltpu.sync_copy(sel_r, sh_selr.at[w])
    pltpu.sync_copy(sel_s, sh_sels.at[w])
    pltpu.sync_copy(sel_l, sh_sell.at[w])
    pltpu.sync_copy(sel_b, sh_selb.at[w])
    pltpu.sync_copy(sel_w, sh_selw.at[w])
    plsc.subcore_barrier()

    pltpu.sync_copy(sh_selr, asel_r)
    pltpu.sync_copy(sh_sels, asel_s)
    pltpu.sync_copy(sh_sell, asel_l)
    pltpu.sync_copy(sh_selb, asel_b)
    pltpu.sync_copy(sh_selw, asel_w)

    neg1f = jnp.full((16,), -1.0, jnp.float32)
    neg1i = jnp.full((16,), -1, jnp.int32)
    for v in range(OSL // 16):
        o_score[pl.ds(v * 16, 16)] = neg1f
        o_label[pl.ds(v * 16, 16)] = neg1i
        o_box[pl.ds(v * 16, 16)] = neg1i
        o_row[pl.ds(v * 16, 16)] = zero16

    slot0 = w * OSL
    for r in range(NW):
        for v in range(SELCAP // 16):
            rks = asel_r[r, pl.ds(v * 16, 16)]
            loc = rks - slot0
            inm = (loc >= 0) & (loc < OSL)
            plsc.store_scatter(o_score, [loc],
                               asel_s[r, pl.ds(v * 16, 16)], mask=inm)
            plsc.store_scatter(o_label, [loc],
                               asel_l[r, pl.ds(v * 16, 16)], mask=inm)
            plsc.store_scatter(o_box, [loc],
                               asel_b[r, pl.ds(v * 16, 16)], mask=inm)
            plsc.store_scatter(o_row, [loc],
                               asel_w[r, pl.ds(v * 16, 16)], mask=inm)

    for j in range(PW // 128):
        for i in range(8):
            p = j * 128 + i * 16 + lane
            slot = p // 12
            rem = p - slot * 12
            rowv = plsc.load_gather(o_row, [slot])
            pidx_v[j, pl.ds(i * 16, 16)] = rowv * 12 + rem

    cps = [pltpu.async_copy(
        poses_hbm.at[pidx_v.at[j]],
        prow_v.at[pl.ds(j * 128, 128)], sem) for j in range(PW // 128)]
    for cp in cps:
        cp.wait()

    def mbody(i, _):
        p = i * 16 + lane
        slot = p // 12
        sv = plsc.load_gather(o_score, [slot])
        pv = prow_v[pl.ds(i * 16, 16)]
        prow_v[pl.ds(i * 16, 16)] = jnp.where(sv > 0.0, pv, -1.0)
        return 0
    lax.fori_loop(0, PW // 16, mbody, 0)

    pltpu.sync_copy(o_score, oscore_hbm.at[pl.ds(slot0, OSL)])
    pltpu.sync_copy(o_label, olabel_hbm.at[pl.ds(slot0, OSL)])
    pltpu.sync_copy(o_box, obox_hbm.at[pl.ds(slot0, OSL)])
    pltpu.sync_copy(prow_v, oposes_hbm.at[pl.ds(w * PW, PW)])


_mesh = plsc.VectorSubcoreMesh(
    core_axis_name="c", subcore_axis_name="s", num_cores=1)

_topk_sc = pl.kernel(
    _body,
    out_type=(
        jax.ShapeDtypeStruct((MAXOUT,), jnp.float32),
        jax.ShapeDtypeStruct((MAXOUT,), jnp.int32),
        jax.ShapeDtypeStruct((NW * PW,), jnp.float32),
        jax.ShapeDtypeStruct((MAXOUT,), jnp.int32),
    ),
    mesh=_mesh,
    compiler_params=pltpu.CompilerParams(needs_layout_passes=False),
    scratch_types=[
        pltpu.VMEM((CHUNK,), jnp.float32),
        pltpu.VMEM((CHUNK,), jnp.int32),
        pltpu.VMEM((NW * 256,), jnp.int32),
        pltpu.VMEM((256,), jnp.int32),
        pltpu.VMEM((NW, 256), jnp.int32),
        pltpu.VMEM((CCAP,), jnp.int32),
        pltpu.VMEM((CCAP,), jnp.int32),
        pltpu.VMEM((NW, CCAP), jnp.int32),
        pltpu.VMEM((NW, CCAP), jnp.int32),
        pltpu.VMEM((NW, CCAP), jnp.int32),
        pltpu.VMEM((CCAP,), jnp.int32),
        pltpu.VMEM((DENSE,), jnp.int32),
        pltpu.VMEM((DENSE,), jnp.int32),
        pltpu.VMEM((SELCAP,), jnp.int32),
        pltpu.VMEM((SELCAP,), jnp.float32),
        pltpu.VMEM((SELCAP,), jnp.int32),
        pltpu.VMEM((SELCAP,), jnp.int32),
        pltpu.VMEM((SELCAP,), jnp.int32),
        pltpu.VMEM((NW, SELCAP), jnp.int32),
        pltpu.VMEM((NW, SELCAP), jnp.float32),
        pltpu.VMEM((NW, SELCAP), jnp.int32),
        pltpu.VMEM((NW, SELCAP), jnp.int32),
        pltpu.VMEM((NW, SELCAP), jnp.int32),
        pltpu.VMEM((OSL,), jnp.float32),
        pltpu.VMEM((OSL,), jnp.int32),
        pltpu.VMEM((OSL,), jnp.int32),
        pltpu.VMEM((OSL,), jnp.int32),
        pltpu.VMEM((PW // 128, 128), jnp.int32),
        pltpu.VMEM((PW,), jnp.float32),
        pltpu.VMEM_SHARED((NW, 256), jnp.int32),
        pltpu.VMEM_SHARED((NW, CCAP), jnp.int32),
        pltpu.VMEM_SHARED((NW, CCAP), jnp.int32),
        pltpu.VMEM_SHARED((NW, CCAP), jnp.int32),
        pltpu.VMEM_SHARED((NW, SELCAP), jnp.int32),
        pltpu.VMEM_SHARED((NW, SELCAP), jnp.float32),
        pltpu.VMEM_SHARED((NW, SELCAP), jnp.int32),
        pltpu.VMEM_SHARED((NW, SELCAP), jnp.int32),
        pltpu.VMEM_SHARED((NW, SELCAP), jnp.int32),
        pltpu.SemaphoreType.DMA,
    ],
)


def kernel(boxes3D, classification, poses, confidence):
    scores = jnp.concatenate(
        [classification.reshape(-1),
         jnp.zeros((NPAD - NFLAT,), jnp.float32)])
    poses_flat = poses.reshape(-1)
    oscore, olabel, oposes, obox = _topk_sc(scores, poses_flat)
    return (oscore[:K], olabel[:K],
            oposes[: K * 12].reshape(K, 12), obox[:K])


# speedup vs baseline: 1.7838x; 1.0286x over previous
import jax
import jax.numpy as jnp
from jax import lax
from jax.experimental import pallas as pl
from jax.experimental.pallas import tpu as pltpu
from jax.experimental.pallas import tpu_sc as plsc

NUM_CLASSES = 15
K = 300
NFLAT = 5000 * NUM_CLASSES
NW = 16
CHUNK = 4704
NPAD = NW * CHUNK
NV = CHUNK // 16
CCAP = 128
CUSE = 64
DENSE = NW * CUSE
SELCAP = 128
MAXOUT = 512
OSL = MAXOUT // NW
PW = OSL * 12
BIAS = 0x3F000000
KEYMAX = 0x7FFFFF


def _popcnt(mask):
    v = plsc.all_reduce_population_count(mask)
    return jnp.max(v) if v.ndim else v


def _suffix_search(gath, kthr):
    z = jnp.zeros((16,), jnp.int32)

    def jbody(jj, carry):
        carryv, bv, mv = carry
        j = 15 - jj
        tot = z
        for r in range(NW):
            tot = tot + gath[pl.ds(r * 256 + j * 16, 16)]
        cs = plsc.cumsum(lax.rev(tot, (0,)))
        s_incl = lax.rev(cs, (0,)) + carryv
        g = s_incl - tot
        ge = g >= kthr
        bv = bv + plsc.all_reduce_population_count(ge)
        mv = jnp.maximum(mv, jnp.where(ge, 0, g))
        carryv = carryv + jnp.max(cs)
        return carryv, bv, mv

    _, bv, mv = lax.fori_loop(0, 16, jbody, (z, z, z))
    return bv, mv


def _histogram(keys_v, hist_f, lane, bucket_fn, mask_fn):
    def zbody(i, _):
        hist_f[pl.ds(i * 16, 16)] = jnp.zeros((16,), jnp.int32)
        return 0
    lax.fori_loop(0, NW * 256 // 16, zbody, 0)
    ones = jnp.full((16,), 1, jnp.int32)
    row = lane * 256

    def hbody(i, _):
        keyv = keys_v[pl.ds(i * 16, 16)]
        plsc.addupdate_scatter(hist_f, [row + bucket_fn(keyv)], ones,
                               mask=mask_fn(keyv))
        return 0
    lax.fori_loop(0, NV, hbody, 0)


def _reduce_hist(hist_f, red_v):
    def rbody(c, _):
        acc = jnp.zeros((16,), jnp.int32)
        for r in range(NW):
            acc = acc + hist_f[pl.ds(r * 256 + c * 16, 16)]
        red_v[pl.ds(c * 16, 16)] = acc
        return 0
    lax.fori_loop(0, 16, rbody, 0)


def _body(scores_hbm, poses_hbm,
          oscore_hbm, olabel_hbm, oposes_hbm, obox_hbm,
          chunk_v, keys_v, hist_f, red_v, gath_v,
          cand_k, cand_x, allk_v, allx_v, cnt2_v, tmp_v,
          dense_k, dense_x,
          sel_r, sel_s, sel_l, sel_b, sel_w,
          asel_r, asel_s, asel_l, asel_b, asel_w,
          o_score, o_label, o_box, o_row, pidx_v, prow_v,
          sh_hist, sh_cnt, sh_ck, sh_cx,
          sh_selr, sh_sels, sh_sell, sh_selb, sh_selw, sem):
    w = lax.axis_index("s")
    lane = lax.iota(jnp.int32, 16)
    base = w * CHUNK
    kvec = jnp.full((16,), K, jnp.int32)
    zero16 = jnp.zeros((16,), jnp.int32)

    pltpu.sync_copy(scores_hbm.at[pl.ds(base, CHUNK)], chunk_v)

    def keybody(i, _):
        sv = chunk_v[pl.ds(i * 16, 16)]
        bits = lax.bitcast_convert_type(sv, jnp.int32)
        validm = sv > 0.5
        keyv = jnp.where(
            validm, jnp.clip(bits - BIAS, 1, KEYMAX), 0)
        keys_v[pl.ds(i * 16, 16)] = keyv
        return 0
    lax.fori_loop(0, NV, keybody, 0)

    _histogram(keys_v, hist_f, lane,
               lambda kv: kv >> 15, lambda kv: kv > 0)
    _reduce_hist(hist_f, red_v)
    pltpu.sync_copy(red_v, sh_hist.at[pl.ds(w * 256, 256)])
    plsc.subcore_barrier()

    pltpu.sync_copy(sh_hist, gath_v)
    b1v, m1v = _suffix_search(gath_v, kvec)

    _histogram(keys_v, hist_f, lane,
               lambda kv: (kv >> 7) & 255,
               lambda kv: (kv > 0) & ((kv >> 15) == b1v))
    _reduce_hist(hist_f, red_v)
    pltpu.sync_copy(red_v, sh_hist.at[pl.ds(w * 256, 256)])
    plsc.subcore_barrier()
    pltpu.sync_copy(sh_hist, gath_v)
    b2v, _ = _suffix_search(gath_v, kvec - m1v)

    ktv = jnp.maximum((b1v << 15) | (b2v << 7), 1)

    def czero(v, _):
        cand_k[pl.ds(v * 16, 16)] = zero16
        cand_x[pl.ds(v * 16, 16)] = zero16
        return 0
    lax.fori_loop(0, CCAP // 16, czero, 0)

    def cbody(i, off):
        keyv = keys_v[pl.ds(i * 16, 16)]
        m = keyv >= ktv
        idxv = base + i * 16 + lane
        offc = jnp.minimum(off, CUSE - 16)
        plsc.store_compressed(cand_k.at[pl.ds(offc, 16)], keyv, mask=m)
        plsc.store_compressed(cand_x.at[pl.ds(offc, 16)], idxv, mask=m)
        return off + _popcnt(m)
    myc = lax.fori_loop(0, NV, cbody, jnp.int32(0))
    myc = jnp.minimum(myc, CUSE)

    mycv = jnp.full((16,), myc, jnp.int32)
    for v in range(CCAP // 16):
        tmp_v[pl.ds(v * 16, 16)] = mycv
    pltpu.sync_copy(tmp_v, sh_cnt.at[pl.ds(w * CCAP, CCAP)])
    pltpu.sync_copy(cand_k, sh_ck.at[pl.ds(w * CCAP, CCAP)])
    pltpu.sync_copy(cand_x, sh_cx.at[pl.ds(w * CCAP, CCAP)])
    plsc.subcore_barrier()

    pltpu.sync_copy(sh_ck, allk_v)
    pltpu.sync_copy(sh_cx, allx_v)
    pltpu.sync_copy(sh_cnt, cnt2_v)

    def dbody(i, ctot):
        r = i // (CUSE // 16)
        v = i - r * (CUSE // 16)
        cntr = jnp.max(cnt2_v[pl.ds(r * CCAP, 16)])
        m = (v * 16 + lane) < cntr
        kk = allk_v[pl.ds(r * CCAP + v * 16, 16)]
        xx = allx_v[pl.ds(r * CCAP + v * 16, 16)]
        offc = jnp.minimum(ctot, DENSE - 16)
        plsc.store_compressed(dense_k.at[pl.ds(offc, 16)], kk, mask=m)
        plsc.store_compressed(dense_x.at[pl.ds(offc, 16)], xx, mask=m)
        return ctot + _popcnt(m)
    ctot = lax.fori_loop(0, NW * (CUSE // 16), dbody, jnp.int32(0))

    own_k = [cand_k[pl.ds(v * 16, 16)] for v in range(CUSE // 16)]
    own_x = [cand_x[pl.ds(v * 16, 16)] for v in range(CUSE // 16)]

    def rjbody(j, ranks):
        jj = jnp.full((16,), j, jnp.int32)
        kj = plsc.load_gather(dense_k, [jj])
        xj = plsc.load_gather(dense_x, [jj])
        out = []
        for v in range(CUSE // 16):
            beat = (kj > own_k[v]) | ((kj == own_k[v]) & (xj < own_x[v]))
            out.append(ranks[v] + jnp.where(beat, 1, 0))
        return tuple(out)
    ranks = lax.fori_loop(0, ctot, rjbody,
                          tuple(zero16 for _ in range(CUSE // 16)))

    def pfill(v, _):
        sel_r[pl.ds(v * 16, 16)] = K + v * 16 + lane
        return 0
    lax.fori_loop(0, SELCAP // 16, pfill, 0)

    selcnt = jnp.int32(0)
    for v in range(CUSE // 16):
        selm = (ranks[v] < kvec) & (own_k[v] > 0)
        sc = jnp.minimum(selcnt, SELCAP - 16)
        scorev = lax.bitcast_convert_type(own_k[v] + BIAS, jnp.float32)
        plsc.store_compressed(sel_r.at[pl.ds(sc, 16)], ranks[v], mask=selm)
        plsc.store_compressed(sel_s.at[pl.ds(sc, 16)], scorev, mask=selm)
        plsc.store_compressed(sel_l.at[pl.ds(sc, 16)],
                              own_x[v] % NUM_CLASSES, mask=selm)
        plsc.store_compressed(sel_b.at[pl.ds(sc, 16)],
                              own_x[v] // NUM_CLASSES, mask=selm)
        plsc.store_compressed(sel_w.at[pl.ds(sc, 16)], own_x[v], mask=selm)
        selcnt = selcnt + _popcnt(selm)

    pltpu.sync_copy(sel_r, sh_selr.at[pl.ds(w * SELCAP, SELCAP)])
    pltpu.sync_copy(sel_s, sh_sels.at[pl.ds(w * SELCAP, SELCAP)])
    pltpu.sync_copy(sel_l, sh_sell.at[pl.ds(w * SELCAP, SELCAP)])
    pltpu.sync_copy(sel_b, sh_selb.at[pl.ds(w * SELCAP, SELCAP)])
    pltpu.sync_copy(sel_w, sh_selw.at[pl.ds(w * SELCAP, SELCAP)])
    plsc.subcore_barrier()

    pltpu.sync_copy(sh_selr, asel_r)
    pltpu.sync_copy(sh_sels, asel_s)
    pltpu.sync_copy(sh_sell, asel_l)
    pltpu.sync_copy(sh_selb, asel_b)
    pltpu.sync_copy(sh_selw, asel_w)

    neg1f = jnp.full((16,), -1.0, jnp.float32)
    neg1i = jnp.full((16,), -1, jnp.int32)
    for v in range(OSL // 16):
        o_score[pl.ds(v * 16, 16)] = neg1f
        o_label[pl.ds(v * 16, 16)] = neg1i
        o_box[pl.ds(v * 16, 16)] = neg1i
        o_row[pl.ds(v * 16, 16)] = zero16

    slot0 = w * OSL

    def abody(t, _):
        rks = asel_r[pl.ds(t * 16, 16)]
        loc = rks - slot0
        inm = (loc >= 0) & (loc < OSL)
        plsc.store_scatter(o_score, [loc],
                           asel_s[pl.ds(t * 16, 16)], mask=inm)
        plsc.store_scatter(o_label, [loc],
                           asel_l[pl.ds(t * 16, 16)], mask=inm)
        plsc.store_scatter(o_box, [loc],
                           asel_b[pl.ds(t * 16, 16)], mask=inm)
        plsc.store_scatter(o_row, [loc],
                           asel_w[pl.ds(t * 16, 16)], mask=inm)
        return 0
    lax.fori_loop(0, NW * SELCAP // 16, abody, 0)

    for j in range(PW // 128):
        for i in range(8):
            p = j * 128 + i * 16 + lane
            slot = p // 12
            rem = p - slot * 12
            rowv = plsc.load_gather(o_row, [slot])
            pidx_v[j, pl.ds(i * 16, 16)] = rowv * 12 + rem

    cps = [pltpu.async_copy(
        poses_hbm.at[pidx_v.at[j]],
        prow_v.at[pl.ds(j * 128, 128)], sem) for j in range(PW // 128)]
    for cp in cps:
        cp.wait()

    def mbody(i, _):
        p = i * 16 + lane
        slot = p // 12
        sv = plsc.load_gather(o_score, [slot])
        pv = prow_v[pl.ds(i * 16, 16)]
        prow_v[pl.ds(i * 16, 16)] = jnp.where(sv > 0.0, pv, -1.0)
        return 0
    lax.fori_loop(0, PW // 16, mbody, 0)

    pltpu.sync_copy(o_score, oscore_hbm.at[pl.ds(slot0, OSL)])
    pltpu.sync_copy(o_label, olabel_hbm.at[pl.ds(slot0, OSL)])
    pltpu.sync_copy(o_box, obox_hbm.at[pl.ds(slot0, OSL)])
    pltpu.sync_copy(prow_v, oposes_hbm.at[pl.ds(w * PW, PW)])


_mesh = plsc.VectorSubcoreMesh(
    core_axis_name="c", subcore_axis_name="s", num_cores=1)

_topk_sc = pl.kernel(
    _body,
    out_type=(
        jax.ShapeDtypeStruct((MAXOUT,), jnp.float32),
        jax.ShapeDtypeStruct((MAXOUT,), jnp.int32),
        jax.ShapeDtypeStruct((NW * PW,), jnp.float32),
        jax.ShapeDtypeStruct((MAXOUT,), jnp.int32),
    ),
    mesh=_mesh,
    compiler_params=pltpu.CompilerParams(needs_layout_passes=False),
    scratch_types=[
        pltpu.VMEM((CHUNK,), jnp.float32),
        pltpu.VMEM((CHUNK,), jnp.int32),
        pltpu.VMEM((NW * 256,), jnp.int32),
        pltpu.VMEM((256,), jnp.int32),
        pltpu.VMEM((NW * 256,), jnp.int32),
        pltpu.VMEM((CCAP,), jnp.int32),
        pltpu.VMEM((CCAP,), jnp.int32),
        pltpu.VMEM((NW * CCAP,), jnp.int32),
        pltpu.VMEM((NW * CCAP,), jnp.int32),
        pltpu.VMEM((NW * CCAP,), jnp.int32),
        pltpu.VMEM((CCAP,), jnp.int32),
        pltpu.VMEM((DENSE,), jnp.int32),
        pltpu.VMEM((DENSE,), jnp.int32),
        pltpu.VMEM((SELCAP,), jnp.int32),
        pltpu.VMEM((SELCAP,), jnp.float32),
        pltpu.VMEM((SELCAP,), jnp.int32),
        pltpu.VMEM((SELCAP,), jnp.int32),
        pltpu.VMEM((SELCAP,), jnp.int32),
        pltpu.VMEM((NW * SELCAP,), jnp.int32),
        pltpu.VMEM((NW * SELCAP,), jnp.float32),
        pltpu.VMEM((NW * SELCAP,), jnp.int32),
        pltpu.VMEM((NW * SELCAP,), jnp.int32),
        pltpu.VMEM((NW * SELCAP,), jnp.int32),
        pltpu.VMEM((OSL,), jnp.float32),
        pltpu.VMEM((OSL,), jnp.int32),
        pltpu.VMEM((OSL,), jnp.int32),
        pltpu.VMEM((OSL,), jnp.int32),
        pltpu.VMEM((PW // 128, 128), jnp.int32),
        pltpu.VMEM((PW,), jnp.float32),
        pltpu.VMEM_SHARED((NW * 256,), jnp.int32),
        pltpu.VMEM_SHARED((NW * CCAP,), jnp.int32),
        pltpu.VMEM_SHARED((NW * CCAP,), jnp.int32),
        pltpu.VMEM_SHARED((NW * CCAP,), jnp.int32),
        pltpu.VMEM_SHARED((NW * SELCAP,), jnp.int32),
        pltpu.VMEM_SHARED((NW * SELCAP,), jnp.float32),
        pltpu.VMEM_SHARED((NW * SELCAP,), jnp.int32),
        pltpu.VMEM_SHARED((NW * SELCAP,), jnp.int32),
        pltpu.VMEM_SHARED((NW * SELCAP,), jnp.int32),
        pltpu.SemaphoreType.DMA,
    ],
)


def kernel(boxes3D, classification, poses, confidence):
    scores = jnp.concatenate(
        [classification.reshape(-1),
         jnp.zeros((NPAD - NFLAT,), jnp.float32)])
    poses_flat = poses.reshape(-1)
    oscore, olabel, oposes, obox = _topk_sc(scores, poses_flat)
    return (oscore[:K], olabel[:K],
            oposes[: K * 12].reshape(K, 12), obox[:K])


# speedup vs baseline: 1.7959x; 1.0068x over previous
import jax
import jax.numpy as jnp
from jax import lax
from jax.experimental import pallas as pl
from jax.experimental.pallas import tpu as pltpu
from jax.experimental.pallas import tpu_sc as plsc

NUM_CLASSES = 15
K = 300
NFLAT = 5000 * NUM_CLASSES
NW = 16
CHUNK = 4704
NPAD = NW * CHUNK
NV = CHUNK // 16
CCAP = 128
CUSE = 64
DENSE = NW * CUSE
SELCAP = 128
MAXOUT = 512
OSL = MAXOUT // NW
PW = OSL * 12
BIAS = 0x3F000000
KEYMAX = 0x7FFFFF


def _popcnt(mask):
    v = plsc.all_reduce_population_count(mask)
    return jnp.max(v) if v.ndim else v


def _suffix_search(gath, kthr):
    z = jnp.zeros((16,), jnp.int32)

    def jbody(jj, carry):
        carryv, bv, mv = carry
        j = 15 - jj
        tot = z
        for r in range(NW):
            tot = tot + gath[pl.ds(r * 256 + j * 16, 16)]
        cs = plsc.cumsum(lax.rev(tot, (0,)))
        s_incl = lax.rev(cs, (0,)) + carryv
        g = s_incl - tot
        ge = g >= kthr
        bv = bv + plsc.all_reduce_population_count(ge)
        mv = jnp.maximum(mv, jnp.where(ge, 0, g))
        carryv = carryv + jnp.max(cs)
        return carryv, bv, mv

    _, bv, mv = lax.fori_loop(0, 16, jbody, (z, z, z))
    return bv, mv


def _histogram(keys_v, hist_f, lane, bucket_fn, mask_fn, zero=False):
    if zero:
        def zbody(i, _):
            hist_f[pl.ds(i * 16, 16)] = jnp.zeros((16,), jnp.int32)
            return 0
        lax.fori_loop(0, NW * 256 // 16, zbody, 0)
    ones = jnp.full((16,), 1, jnp.int32)
    row = lane * 256

    def hbody(i, _):
        keyv = keys_v[pl.ds(i * 16, 16)]
        plsc.addupdate_scatter(hist_f, [row + bucket_fn(keyv)], ones,
                               mask=mask_fn(keyv))
        return 0
    lax.fori_loop(0, NV, hbody, 0)


def _reduce_hist(hist_f, red_v):
    z = jnp.zeros((16,), jnp.int32)

    def rbody(c, _):
        acc = z
        for r in range(NW):
            acc = acc + hist_f[pl.ds(r * 256 + c * 16, 16)]
            hist_f[pl.ds(r * 256 + c * 16, 16)] = z
        red_v[pl.ds(c * 16, 16)] = acc
        return 0
    lax.fori_loop(0, 16, rbody, 0)


def _body(scores_hbm, poses_hbm,
          oscore_hbm, olabel_hbm, oposes_hbm, obox_hbm,
          chunk_v, keys_v, hist_f, red_v, gath_v,
          cand_k, cand_x, allk_v, allx_v, cnt2_v, tmp_v,
          dense_k, dense_x,
          sel_r, sel_s, sel_l, sel_b, sel_w,
          asel_r, asel_s, asel_l, asel_b, asel_w,
          o_score, o_label, o_box, o_row, pidx_v, prow_v,
          sh_hist, sh_cnt, sh_ck, sh_cx,
          sh_selr, sh_sels, sh_sell, sh_selb, sh_selw, sem):
    w = lax.axis_index("s")
    lane = lax.iota(jnp.int32, 16)
    base = w * CHUNK
    kvec = jnp.full((16,), K, jnp.int32)
    zero16 = jnp.zeros((16,), jnp.int32)

    pltpu.sync_copy(scores_hbm.at[pl.ds(base, CHUNK)], chunk_v)

    def zbody(i, _):
        hist_f[pl.ds(i * 16, 16)] = zero16
        return 0
    lax.fori_loop(0, NW * 256 // 16, zbody, 0)

    ones16 = jnp.full((16,), 1, jnp.int32)
    row16 = lane * 256

    def keyhist(i, _):
        sv = chunk_v[pl.ds(i * 16, 16)]
        bits = lax.bitcast_convert_type(sv, jnp.int32)
        validm = sv > 0.5
        keyv = jnp.where(
            validm, jnp.clip(bits - BIAS, 1, KEYMAX), 0)
        keys_v[pl.ds(i * 16, 16)] = keyv
        plsc.addupdate_scatter(hist_f, [row16 + (keyv >> 15)], ones16,
                               mask=keyv > 0)
        return 0
    lax.fori_loop(0, NV, keyhist, 0)

    _reduce_hist(hist_f, red_v)
    pltpu.sync_copy(red_v, sh_hist.at[pl.ds(w * 256, 256)])
    plsc.subcore_barrier()

    pltpu.sync_copy(sh_hist, gath_v)
    b1v, m1v = _suffix_search(gath_v, kvec)

    _histogram(keys_v, hist_f, lane,
               lambda kv: (kv >> 7) & 255,
               lambda kv: (kv > 0) & ((kv >> 15) == b1v))
    _reduce_hist(hist_f, red_v)
    pltpu.sync_copy(red_v, sh_hist.at[pl.ds(w * 256, 256)])
    plsc.subcore_barrier()
    pltpu.sync_copy(sh_hist, gath_v)
    b2v, _ = _suffix_search(gath_v, kvec - m1v)

    ktv = jnp.maximum((b1v << 15) | (b2v << 7), 1)

    def czero(v, _):
        cand_k[pl.ds(v * 16, 16)] = zero16
        cand_x[pl.ds(v * 16, 16)] = zero16
        return 0
    lax.fori_loop(0, CCAP // 16, czero, 0)

    def cbody(i, off):
        keyv = keys_v[pl.ds(i * 16, 16)]
        m = keyv >= ktv
        idxv = base + i * 16 + lane
        offc = jnp.minimum(off, CUSE - 16)
        plsc.store_compressed(cand_k.at[pl.ds(offc, 16)], keyv, mask=m)
        plsc.store_compressed(cand_x.at[pl.ds(offc, 16)], idxv, mask=m)
        return off + _popcnt(m)
    myc = lax.fori_loop(0, NV, cbody, jnp.int32(0))
    myc = jnp.minimum(myc, CUSE)

    mycv = jnp.full((16,), myc, jnp.int32)
    for v in range(CCAP // 16):
        tmp_v[pl.ds(v * 16, 16)] = mycv
    cps = [pltpu.async_copy(tmp_v, sh_cnt.at[pl.ds(w * CCAP, CCAP)], sem),
           pltpu.async_copy(cand_k, sh_ck.at[pl.ds(w * CCAP, CCAP)], sem),
           pltpu.async_copy(cand_x, sh_cx.at[pl.ds(w * CCAP, CCAP)], sem)]
    for cp in cps:
        cp.wait()
    plsc.subcore_barrier()

    cps = [pltpu.async_copy(sh_ck, allk_v, sem),
           pltpu.async_copy(sh_cx, allx_v, sem),
           pltpu.async_copy(sh_cnt, cnt2_v, sem)]
    for cp in cps:
        cp.wait()

    def dbody(i, ctot):
        r = i // (CUSE // 16)
        v = i - r * (CUSE // 16)
        cntr = jnp.max(cnt2_v[pl.ds(r * CCAP, 16)])
        m = (v * 16 + lane) < cntr
        kk = allk_v[pl.ds(r * CCAP + v * 16, 16)]
        xx = allx_v[pl.ds(r * CCAP + v * 16, 16)]
        offc = jnp.minimum(ctot, DENSE - 16)
        plsc.store_compressed(dense_k.at[pl.ds(offc, 16)], kk, mask=m)
        plsc.store_compressed(dense_x.at[pl.ds(offc, 16)], xx, mask=m)
        return ctot + _popcnt(m)
    ctot = lax.fori_loop(0, NW * (CUSE // 16), dbody, jnp.int32(0))

    own_k = [cand_k[pl.ds(v * 16, 16)] for v in range(CUSE // 16)]
    own_x = [cand_x[pl.ds(v * 16, 16)] for v in range(CUSE // 16)]

    def rjbody(j, ranks):
        jj = jnp.full((16,), j, jnp.int32)
        kj = plsc.load_gather(dense_k, [jj])
        xj = plsc.load_gather(dense_x, [jj])
        out = []
        for v in range(CUSE // 16):
            beat = (kj > own_k[v]) | ((kj == own_k[v]) & (xj < own_x[v]))
            out.append(ranks[v] + jnp.where(beat, 1, 0))
        return tuple(out)
    ranks = lax.fori_loop(0, ctot, rjbody,
                          tuple(zero16 for _ in range(CUSE // 16)))

    def pfill(v, _):
        sel_r[pl.ds(v * 16, 16)] = K + v * 16 + lane
        return 0
    lax.fori_loop(0, SELCAP // 16, pfill, 0)

    selcnt = jnp.int32(0)
    for v in range(CUSE // 16):
        selm = (ranks[v] < kvec) & (own_k[v] > 0)
        sc = jnp.minimum(selcnt, SELCAP - 16)
        scorev = lax.bitcast_convert_type(own_k[v] + BIAS, jnp.float32)
        plsc.store_compressed(sel_r.at[pl.ds(sc, 16)], ranks[v], mask=selm)
        plsc.store_compressed(sel_s.at[pl.ds(sc, 16)], scorev, mask=selm)
        plsc.store_compressed(sel_l.at[pl.ds(sc, 16)],
                              own_x[v] % NUM_CLASSES, mask=selm)
        plsc.store_compressed(sel_b.at[pl.ds(sc, 16)],
                              own_x[v] // NUM_CLASSES, mask=selm)
        plsc.store_compressed(sel_w.at[pl.ds(sc, 16)], own_x[v], mask=selm)
        selcnt = selcnt + _popcnt(selm)

    cps = [pltpu.async_copy(sel_r, sh_selr.at[pl.ds(w * SELCAP, SELCAP)], sem),
           pltpu.async_copy(sel_s, sh_sels.at[pl.ds(w * SELCAP, SELCAP)], sem),
           pltpu.async_copy(sel_l, sh_sell.at[pl.ds(w * SELCAP, SELCAP)], sem),
           pltpu.async_copy(sel_b, sh_selb.at[pl.ds(w * SELCAP, SELCAP)], sem),
           pltpu.async_copy(sel_w, sh_selw.at[pl.ds(w * SELCAP, SELCAP)], sem)]
    for cp in cps:
        cp.wait()
    plsc.subcore_barrier()

    cps = [pltpu.async_copy(sh_selr, asel_r, sem),
           pltpu.async_copy(sh_sels, asel_s, sem),
           pltpu.async_copy(sh_sell, asel_l, sem),
           pltpu.async_copy(sh_selb, asel_b, sem),
           pltpu.async_copy(sh_selw, asel_w, sem)]
    for cp in cps:
        cp.wait()

    neg1f = jnp.full((16,), -1.0, jnp.float32)
    neg1i = jnp.full((16,), -1, jnp.int32)
    for v in range(OSL // 16):
        o_score[pl.ds(v * 16, 16)] = neg1f
        o_label[pl.ds(v * 16, 16)] = neg1i
        o_box[pl.ds(v * 16, 16)] = neg1i
        o_row[pl.ds(v * 16, 16)] = zero16

    slot0 = w * OSL

    def abody(t, _):
        rks = asel_r[pl.ds(t * 16, 16)]
        loc = rks - slot0
        inm = (loc >= 0) & (loc < OSL)
        plsc.store_scatter(o_score, [loc],
                           asel_s[pl.ds(t * 16, 16)], mask=inm)
        plsc.store_scatter(o_label, [loc],
                           asel_l[pl.ds(t * 16, 16)], mask=inm)
        plsc.store_scatter(o_box, [loc],
                           asel_b[pl.ds(t * 16, 16)], mask=inm)
        plsc.store_scatter(o_row, [loc],
                           asel_w[pl.ds(t * 16, 16)], mask=inm)
        return 0
    lax.fori_loop(0, NW * SELCAP // 16, abody, 0)

    for j in range(PW // 128):
        for i in range(8):
            p = j * 128 + i * 16 + lane
            slot = p // 12
            rem = p - slot * 12
            rowv = plsc.load_gather(o_row, [slot])
            pidx_v[j, pl.ds(i * 16, 16)] = rowv * 12 + rem

    cps = [pltpu.async_copy(
        poses_hbm.at[pidx_v.at[j]],
        prow_v.at[pl.ds(j * 128, 128)], sem) for j in range(PW // 128)]
    for cp in cps:
        cp.wait()

    def mbody(i, _):
        p = i * 16 + lane
        slot = p // 12
        sv = plsc.load_gather(o_score, [slot])
        pv = prow_v[pl.ds(i * 16, 16)]
        prow_v[pl.ds(i * 16, 16)] = jnp.where(sv > 0.0, pv, -1.0)
        return 0
    lax.fori_loop(0, PW // 16, mbody, 0)

    cps = [pltpu.async_copy(o_score, oscore_hbm.at[pl.ds(slot0, OSL)], sem),
           pltpu.async_copy(o_label, olabel_hbm.at[pl.ds(slot0, OSL)], sem),
           pltpu.async_copy(o_box, obox_hbm.at[pl.ds(slot0, OSL)], sem),
           pltpu.async_copy(prow_v, oposes_hbm.at[pl.ds(w * PW, PW)], sem)]
    for cp in cps:
        cp.wait()


_mesh = plsc.VectorSubcoreMesh(
    core_axis_name="c", subcore_axis_name="s", num_cores=1)

_topk_sc = pl.kernel(
    _body,
    out_type=(
        jax.ShapeDtypeStruct((MAXOUT,), jnp.float32),
        jax.ShapeDtypeStruct((MAXOUT,), jnp.int32),
        jax.ShapeDtypeStruct((NW * PW,), jnp.float32),
        jax.ShapeDtypeStruct((MAXOUT,), jnp.int32),
    ),
    mesh=_mesh,
    compiler_params=pltpu.CompilerParams(needs_layout_passes=False),
    scratch_types=[
        pltpu.VMEM((CHUNK,), jnp.float32),
        pltpu.VMEM((CHUNK,), jnp.int32),
        pltpu.VMEM((NW * 256,), jnp.int32),
        pltpu.VMEM((256,), jnp.int32),
        pltpu.VMEM((NW * 256,), jnp.int32),
        pltpu.VMEM((CCAP,), jnp.int32),
        pltpu.VMEM((CCAP,), jnp.int32),
        pltpu.VMEM((NW * CCAP,), jnp.int32),
        pltpu.VMEM((NW * CCAP,), jnp.int32),
        pltpu.VMEM((NW * CCAP,), jnp.int32),
        pltpu.VMEM((CCAP,), jnp.int32),
        pltpu.VMEM((DENSE,), jnp.int32),
        pltpu.VMEM((DENSE,), jnp.int32),
        pltpu.VMEM((SELCAP,), jnp.int32),
        pltpu.VMEM((SELCAP,), jnp.float32),
        pltpu.VMEM((SELCAP,), jnp.int32),
        pltpu.VMEM((SELCAP,), jnp.int32),
        pltpu.VMEM((SELCAP,), jnp.int32),
        pltpu.VMEM((NW * SELCAP,), jnp.int32),
        pltpu.VMEM((NW * SELCAP,), jnp.float32),
        pltpu.VMEM((NW * SELCAP,), jnp.int32),
        pltpu.VMEM((NW * SELCAP,), jnp.int32),
        pltpu.VMEM((NW * SELCAP,), jnp.int32),
        pltpu.VMEM((OSL,), jnp.float32),
        pltpu.VMEM((OSL,), jnp.int32),
        pltpu.VMEM((OSL,), jnp.int32),
        pltpu.VMEM((OSL,), jnp.int32),
        pltpu.VMEM((PW // 128, 128), jnp.int32),
        pltpu.VMEM((PW,), jnp.float32),
        pltpu.VMEM_SHARED((NW * 256,), jnp.int32),
        pltpu.VMEM_SHARED((NW * CCAP,), jnp.int32),
        pltpu.VMEM_SHARED((NW * CCAP,), jnp.int32),
        pltpu.VMEM_SHARED((NW * CCAP,), jnp.int32),
        pltpu.VMEM_SHARED((NW * SELCAP,), jnp.int32),
        pltpu.VMEM_SHARED((NW * SELCAP,), jnp.float32),
        pltpu.VMEM_SHARED((NW * SELCAP,), jnp.int32),
        pltpu.VMEM_SHARED((NW * SELCAP,), jnp.int32),
        pltpu.VMEM_SHARED((NW * SELCAP,), jnp.int32),
        pltpu.SemaphoreType.DMA,
    ],
)


def kernel(boxes3D, classification, poses, confidence):
    scores = jnp.concatenate(
        [classification.reshape(-1),
         jnp.zeros((NPAD - NFLAT,), jnp.float32)])
    poses_flat = poses.reshape(-1)
    oscore, olabel, oposes, obox = _topk_sc(scores, poses_flat)
    return (oscore[:K], olabel[:K],
            oposes[: K * 12].reshape(K, 12), obox[:K])


# speedup vs baseline: 1.8805x; 1.0471x over previous
import jax
import jax.numpy as jnp
from jax import lax
from jax.experimental import pallas as pl
from jax.experimental.pallas import tpu as pltpu
from jax.experimental.pallas import tpu_sc as plsc

NUM_CLASSES = 15
K = 300
NFLAT = 5000 * NUM_CLASSES
NW = 16
CHUNK = 4704
NPAD = NW * CHUNK
NV = CHUNK // 16
CCAP = 128
CUSE = 64
DENSE = NW * CUSE
SELCAP = 128
MAXOUT = 512
OSL = MAXOUT // NW
PW = OSL * 12
BIAS = 0x3F000000
KEYMAX = 0x7FFFFF


def _popcnt(mask):
    v = plsc.all_reduce_population_count(mask)
    return jnp.max(v) if v.ndim else v


def _suffix_search(gath, kthr):
    z = jnp.zeros((16,), jnp.int32)

    def jbody(jj, carry):
        carryv, bv, mv = carry
        j = 15 - jj
        tot = z
        for r in range(NW):
            tot = tot + gath[pl.ds(r * 256 + j * 16, 16)]
        cs = plsc.cumsum(lax.rev(tot, (0,)))
        s_incl = lax.rev(cs, (0,)) + carryv
        g = s_incl - tot
        ge = g >= kthr
        bv = bv + plsc.all_reduce_population_count(ge)
        mv = jnp.maximum(mv, jnp.where(ge, 0, g))
        carryv = carryv + jnp.max(cs)
        return carryv, bv, mv

    _, bv, mv = lax.fori_loop(0, 16, jbody, (z, z, z))
    return bv, mv


def _reduce_hist(hist_f, red_v):
    z = jnp.zeros((16,), jnp.int32)

    def rbody(c, _):
        acc = z
        for r in range(NW):
            acc = acc + hist_f[pl.ds(r * 256 + c * 16, 16)]
        red_v[pl.ds(c * 16, 16)] = acc
        return 0
    lax.fori_loop(0, 16, rbody, 0)


def _body(scores_hbm, poses_hbm,
          oscore_hbm, olabel_hbm, oposes_hbm, obox_hbm,
          chunk_v, keys_v, hist_f, red_v, gath_v,
          cand_k, cand_x, allk_v, allx_v, cnt2_v, tmp_v,
          dense_k, dense_x,
          sel_r, sel_s, sel_l, sel_b, sel_w,
          asel_r, asel_s, asel_l, asel_b, asel_w,
          o_score, o_label, o_box, o_row, pidx_v, prow_v,
          sh_hist, sh_cnt, sh_ck, sh_cx,
          sh_selr, sh_sels, sh_sell, sh_selb, sh_selw, sem):
    w = lax.axis_index("s")
    lane = lax.iota(jnp.int32, 16)
    base = w * CHUNK
    kvec = jnp.full((16,), K, jnp.int32)
    zero16 = jnp.zeros((16,), jnp.int32)

    pltpu.sync_copy(scores_hbm.at[pl.ds(base, CHUNK)], chunk_v)

    def zbody(i, _):
        hist_f[pl.ds(i * 16, 16)] = zero16
        return 0
    lax.fori_loop(0, NW * 256 // 16, zbody, 0)

    ones16 = jnp.full((16,), 1, jnp.int32)
    row16 = lane * 256

    def keyhist(i, _):
        sv = chunk_v[pl.ds(i * 16, 16)]
        bits = lax.bitcast_convert_type(sv, jnp.int32)
        validm = sv > 0.5
        keyv = jnp.where(
            validm, jnp.clip(bits - BIAS, 1, KEYMAX), 0)
        keys_v[pl.ds(i * 16, 16)] = keyv
        plsc.addupdate_scatter(hist_f, [row16 + (keyv >> 15)], ones16,
                               mask=keyv > 0)
        return 0
    lax.fori_loop(0, NV, keyhist, 0)

    _reduce_hist(hist_f, red_v)
    pltpu.sync_copy(red_v, sh_hist.at[pl.ds(w * 256, 256)])
    plsc.subcore_barrier()

    pltpu.sync_copy(sh_hist, gath_v)
    b1v, m1v = _suffix_search(gath_v, kvec)

    del m1v
    ktv = jnp.maximum(b1v << 15, 1)

    def czero(v, _):
        cand_k[pl.ds(v * 16, 16)] = zero16
        cand_x[pl.ds(v * 16, 16)] = zero16
        return 0
    lax.fori_loop(0, CCAP // 16, czero, 0)

    def cbody(i, off):
        keyv = keys_v[pl.ds(i * 16, 16)]
        m = keyv >= ktv
        idxv = base + i * 16 + lane
        offc = jnp.minimum(off, CUSE - 16)
        plsc.store_compressed(cand_k.at[pl.ds(offc, 16)], keyv, mask=m)
        plsc.store_compressed(cand_x.at[pl.ds(offc, 16)], idxv, mask=m)
        return off + _popcnt(m)
    myc = lax.fori_loop(0, NV, cbody, jnp.int32(0))
    myc = jnp.minimum(myc, CUSE)

    mycv = jnp.full((16,), myc, jnp.int32)
    for v in range(CCAP // 16):
        tmp_v[pl.ds(v * 16, 16)] = mycv
    cps = [pltpu.async_copy(tmp_v, sh_cnt.at[pl.ds(w * CCAP, CCAP)], sem),
           pltpu.async_copy(cand_k, sh_ck.at[pl.ds(w * CCAP, CCAP)], sem),
           pltpu.async_copy(cand_x, sh_cx.at[pl.ds(w * CCAP, CCAP)], sem)]
    for cp in cps:
        cp.wait()
    plsc.subcore_barrier()

    cps = [pltpu.async_copy(sh_ck, allk_v, sem),
           pltpu.async_copy(sh_cx, allx_v, sem),
           pltpu.async_copy(sh_cnt, cnt2_v, sem)]
    for cp in cps:
        cp.wait()

    def dbody(i, ctot):
        r = i // (CUSE // 16)
        v = i - r * (CUSE // 16)
        cntr = jnp.max(cnt2_v[pl.ds(r * CCAP, 16)])
        m = (v * 16 + lane) < cntr
        kk = allk_v[pl.ds(r * CCAP + v * 16, 16)]
        xx = allx_v[pl.ds(r * CCAP + v * 16, 16)]
        offc = jnp.minimum(ctot, DENSE - 16)
        plsc.store_compressed(dense_k.at[pl.ds(offc, 16)], kk, mask=m)
        plsc.store_compressed(dense_x.at[pl.ds(offc, 16)], xx, mask=m)
        return ctot + _popcnt(m)
    ctot = lax.fori_loop(0, NW * (CUSE // 16), dbody, jnp.int32(0))

    own_k = [cand_k[pl.ds(v * 16, 16)] for v in range(CUSE // 16)]
    own_x = [cand_x[pl.ds(v * 16, 16)] for v in range(CUSE // 16)]

    def rjbody(j, ranks):
        jj = jnp.full((16,), j, jnp.int32)
        kj = plsc.load_gather(dense_k, [jj])
        xj = plsc.load_gather(dense_x, [jj])
        out = []
        for v in range(CUSE // 16):
            beat = (kj > own_k[v]) | ((kj == own_k[v]) & (xj < own_x[v]))
            out.append(ranks[v] + jnp.where(beat, 1, 0))
        return tuple(out)
    ranks = lax.fori_loop(0, ctot, rjbody,
                          tuple(zero16 for _ in range(CUSE // 16)))

    def pfill(v, _):
        sel_r[pl.ds(v * 16, 16)] = K + v * 16 + lane
        return 0
    lax.fori_loop(0, SELCAP // 16, pfill, 0)

    selcnt = jnp.int32(0)
    for v in range(CUSE // 16):
        selm = (ranks[v] < kvec) & (own_k[v] > 0)
        sc = jnp.minimum(selcnt, SELCAP - 16)
        scorev = lax.bitcast_convert_type(own_k[v] + BIAS, jnp.float32)
        plsc.store_compressed(sel_r.at[pl.ds(sc, 16)], ranks[v], mask=selm)
        plsc.store_compressed(sel_s.at[pl.ds(sc, 16)], scorev, mask=selm)
        plsc.store_compressed(sel_l.at[pl.ds(sc, 16)],
                              own_x[v] % NUM_CLASSES, mask=selm)
        plsc.store_compressed(sel_b.at[pl.ds(sc, 16)],
                              own_x[v] // NUM_CLASSES, mask=selm)
        plsc.store_compressed(sel_w.at[pl.ds(sc, 16)], own_x[v], mask=selm)
        selcnt = selcnt + _popcnt(selm)

    cps = [pltpu.async_copy(sel_r, sh_selr.at[pl.ds(w * SELCAP, SELCAP)], sem),
           pltpu.async_copy(sel_s, sh_sels.at[pl.ds(w * SELCAP, SELCAP)], sem),
           pltpu.async_copy(sel_l, sh_sell.at[pl.ds(w * SELCAP, SELCAP)], sem),
           pltpu.async_copy(sel_b, sh_selb.at[pl.ds(w * SELCAP, SELCAP)], sem),
           pltpu.async_copy(sel_w, sh_selw.at[pl.ds(w * SELCAP, SELCAP)], sem)]
    for cp in cps:
        cp.wait()
    plsc.subcore_barrier()

    cps = [pltpu.async_copy(sh_selr, asel_r, sem),
           pltpu.async_copy(sh_sels, asel_s, sem),
           pltpu.async_copy(sh_sell, asel_l, sem),
           pltpu.async_copy(sh_selb, asel_b, sem),
           pltpu.async_copy(sh_selw, asel_w, sem)]
    for cp in cps:
        cp.wait()

    neg1f = jnp.full((16,), -1.0, jnp.float32)
    neg1i = jnp.full((16,), -1, jnp.int32)
    for v in range(OSL // 16):
        o_score[pl.ds(v * 16, 16)] = neg1f
        o_label[pl.ds(v * 16, 16)] = neg1i
        o_box[pl.ds(v * 16, 16)] = neg1i
        o_row[pl.ds(v * 16, 16)] = zero16

    slot0 = w * OSL

    def abody(t, _):
        rks = asel_r[pl.ds(t * 16, 16)]
        loc = rks - slot0
        inm = (loc >= 0) & (loc < OSL)
        plsc.store_scatter(o_score, [loc],
                           asel_s[pl.ds(t * 16, 16)], mask=inm)
        plsc.store_scatter(o_label, [loc],
                           asel_l[pl.ds(t * 16, 16)], mask=inm)
        plsc.store_scatter(o_box, [loc],
                           asel_b[pl.ds(t * 16, 16)], mask=inm)
        plsc.store_scatter(o_row, [loc],
                           asel_w[pl.ds(t * 16, 16)], mask=inm)
        return 0
    lax.fori_loop(0, NW * SELCAP // 16, abody, 0)

    for j in range(PW // 128):
        for i in range(8):
            p = j * 128 + i * 16 + lane
            slot = p // 12
            rem = p - slot * 12
            rowv = plsc.load_gather(o_row, [slot])
            pidx_v[j, pl.ds(i * 16, 16)] = rowv * 12 + rem

    cps = [pltpu.async_copy(
        poses_hbm.at[pidx_v.at[j]],
        prow_v.at[pl.ds(j * 128, 128)], sem) for j in range(PW // 128)]
    for cp in cps:
        cp.wait()

    def mbody(i, _):
        p = i * 16 + lane
        slot = p // 12
        sv = plsc.load_gather(o_score, [slot])
        pv = prow_v[pl.ds(i * 16, 16)]
        prow_v[pl.ds(i * 16, 16)] = jnp.where(sv > 0.0, pv, -1.0)
        return 0
    lax.fori_loop(0, PW // 16, mbody, 0)

    cps = [pltpu.async_copy(o_score, oscore_hbm.at[pl.ds(slot0, OSL)], sem),
           pltpu.async_copy(o_label, olabel_hbm.at[pl.ds(slot0, OSL)], sem),
           pltpu.async_copy(o_box, obox_hbm.at[pl.ds(slot0, OSL)], sem),
           pltpu.async_copy(prow_v, oposes_hbm.at[pl.ds(w * PW, PW)], sem)]
    for cp in cps:
        cp.wait()


_mesh = plsc.VectorSubcoreMesh(
    core_axis_name="c", subcore_axis_name="s", num_cores=1)

_topk_sc = pl.kernel(
    _body,
    out_type=(
        jax.ShapeDtypeStruct((MAXOUT,), jnp.float32),
        jax.ShapeDtypeStruct((MAXOUT,), jnp.int32),
        jax.ShapeDtypeStruct((NW * PW,), jnp.float32),
        jax.ShapeDtypeStruct((MAXOUT,), jnp.int32),
    ),
    mesh=_mesh,
    compiler_params=pltpu.CompilerParams(needs_layout_passes=False),
    scratch_types=[
        pltpu.VMEM((CHUNK,), jnp.float32),
        pltpu.VMEM((CHUNK,), jnp.int32),
        pltpu.VMEM((NW * 256,), jnp.int32),
        pltpu.VMEM((256,), jnp.int32),
        pltpu.VMEM((NW * 256,), jnp.int32),
        pltpu.VMEM((CCAP,), jnp.int32),
        pltpu.VMEM((CCAP,), jnp.int32),
        pltpu.VMEM((NW * CCAP,), jnp.int32),
        pltpu.VMEM((NW * CCAP,), jnp.int32),
        pltpu.VMEM((NW * CCAP,), jnp.int32),
        pltpu.VMEM((CCAP,), jnp.int32),
        pltpu.VMEM((DENSE,), jnp.int32),
        pltpu.VMEM((DENSE,), jnp.int32),
        pltpu.VMEM((SELCAP,), jnp.int32),
        pltpu.VMEM((SELCAP,), jnp.float32),
        pltpu.VMEM((SELCAP,), jnp.int32),
        pltpu.VMEM((SELCAP,), jnp.int32),
        pltpu.VMEM((SELCAP,), jnp.int32),
        pltpu.VMEM((NW * SELCAP,), jnp.int32),
        pltpu.VMEM((NW * SELCAP,), jnp.float32),
        pltpu.VMEM((NW * SELCAP,), jnp.int32),
        pltpu.VMEM((NW * SELCAP,), jnp.int32),
        pltpu.VMEM((NW * SELCAP,), jnp.int32),
        pltpu.VMEM((OSL,), jnp.float32),
        pltpu.VMEM((OSL,), jnp.int32),
        pltpu.VMEM((OSL,), jnp.int32),
        pltpu.VMEM((OSL,), jnp.int32),
        pltpu.VMEM((PW // 128, 128), jnp.int32),
        pltpu.VMEM((PW,), jnp.float32),
        pltpu.VMEM_SHARED((NW * 256,), jnp.int32),
        pltpu.VMEM_SHARED((NW * CCAP,), jnp.int32),
        pltpu.VMEM_SHARED((NW * CCAP,), jnp.int32),
        pltpu.VMEM_SHARED((NW * CCAP,), jnp.int32),
        pltpu.VMEM_SHARED((NW * SELCAP,), jnp.int32),
        pltpu.VMEM_SHARED((NW * SELCAP,), jnp.float32),
        pltpu.VMEM_SHARED((NW * SELCAP,), jnp.int32),
        pltpu.VMEM_SHARED((NW * SELCAP,), jnp.int32),
        pltpu.VMEM_SHARED((NW * SELCAP,), jnp.int32),
        pltpu.SemaphoreType.DMA,
    ],
)


def kernel(boxes3D, classification, poses, confidence):
    scores = jnp.concatenate(
        [classification.reshape(-1),
         jnp.zeros((NPAD - NFLAT,), jnp.float32)])
    poses_flat = poses.reshape(-1)
    oscore, olabel, oposes, obox = _topk_sc(scores, poses_flat)
    return (oscore[:K], olabel[:K],
            oposes[: K * 12].reshape(K, 12), obox[:K])
